# baseline (speedup 1.0000x reference)
import functools

import jax
import jax.numpy as jnp
from jax import lax
from jax.experimental import pallas as pl
from jax.experimental.pallas import tpu as pltpu
from jax.experimental.pallas import tpu_sc as plsc

S = 2048
B = 4
D = 1024
N_TOK = S * B
L = 16
NC = 2
NS = 16
NW = NC * NS
TOK_PER_W = N_TOK // NW
C = 32
NCH = TOK_PER_W // C
PK = C // 4
NG = C // L
PC = C // B
UNR = 8
NJ = (D // L) // UNR

_MESH = plsc.VectorSubcoreMesh(core_axis_name="c", subcore_axis_name="s")


@functools.partial(
    pl.kernel,
    mesh=_MESH,
    compiler_params=pltpu.CompilerParams(needs_layout_passes=False),
    out_type=jax.ShapeDtypeStruct((S, B, D), jnp.float32),
    scratch_types=[
        pltpu.VMEM((NCH, C), jnp.int32),
        pltpu.VMEM((3, C, D), jnp.float32),
        pltpu.VMEM((2, PC, D), jnp.float32),
        pltpu.VMEM((D,), jnp.float32),
        pltpu.VMEM((D,), jnp.float32),
        pltpu.VMEM((C, L), jnp.float32),
        pltpu.VMEM((C, L), jnp.float32),
        pltpu.VMEM((NG, L), jnp.float32),
        pltpu.VMEM((NG, L), jnp.float32),
        pltpu.SemaphoreType.DMA,
        pltpu.SemaphoreType.DMA,
        pltpu.SemaphoreType.DMA,
        pltpu.SemaphoreType.DMA,
        pltpu.SemaphoreType.DMA,
        pltpu.SemaphoreType.DMA,
        pltpu.SemaphoreType.DMA,
        pltpu.SemaphoreType.DMA,
    ],
)
def _emb_ln(x_hbm, tok_hbm, pos_hbm, gamma_hbm, beta_hbm, out_hbm,
            idx_v, h_v, p_v, gam_v, bet_v, s_v, q_v, mu_v, rs_v,
            g0, g1, g2, p0, p1, w0, w1, w2):
    wid = lax.axis_index("s") * NC + lax.axis_index("c")
    base = wid * TOK_PER_W
    gsem = (g0, g1, g2)
    psem = (p0, p1)
    wsem = (w0, w1, w2)

    pltpu.sync_copy(gamma_hbm, gam_v)
    pltpu.sync_copy(beta_hbm, bet_v)
    pltpu.sync_copy(x_hbm.at[wid], idx_v)

    def issue(c):
        hs = c % 3
        ps = c % 2
        g = pltpu.async_copy(tok_hbm.at[idx_v.at[c]], h_v.at[hs], gsem[hs])
        pb = pl.multiple_of((base + c * C) // B, PC)
        p = pltpu.async_copy(pos_hbm.at[pl.ds(pb, PC)], p_v.at[ps], psem[ps])
        return g, p

    def compute_chunk(hs, ps):
        hb = h_v.at[hs]
        pb = p_v.at[ps]

        def pack_a(p):
            def jbody(j, accs):
                accs = list(accs)
                off = j * L
                pv = pb[p, pl.ds(off, L)]
                for r in range(4):
                    row = p * 4 + r
                    h = hb[row, pl.ds(off, L)] + pv
                    hb[row, pl.ds(off, L)] = h
                    accs[r] = accs[r] + h
                    accs[4 + r] = accs[4 + r] + h * h
                return tuple(accs)

            z = jnp.zeros((L,), jnp.float32)
            accs = plsc.parallel_loop(
                0, D // L, unroll=UNR, carry=(z,) * 8)(jbody)
            for r in range(4):
                s_v[p * 4 + r, :] = accs[r]
                q_v[p * 4 + r, :] = accs[4 + r]

        plsc.parallel_loop(0, PK)(pack_a)

        iota = lax.iota(jnp.int32, L)
        for g in range(NG):
            rows = g * L + iota
            tot = None
            tot2 = None
            for k in range(L):
                kk = jnp.full((L,), k, jnp.int32)
                cs = plsc.load_gather(s_v, [rows, kk])
                cq = plsc.load_gather(q_v, [rows, kk])
                tot = cs if tot is None else tot + cs
                tot2 = cq if tot2 is None else tot2 + cq
            mean = tot * (1.0 / D)
            var = tot2 * (1.0 / D) - mean * mean
            xe = var + 1e-5
            iv = lax.bitcast_convert_type(xe, jnp.int32)
            y = lax.bitcast_convert_type(
                jnp.full((L,), 0x5F3759DF, jnp.int32)
                - lax.shift_right_logical(iv, 1),
                jnp.float32)
            for _ in range(3):
                y = y * (1.5 - 0.5 * xe * y * y)
            mu_v[g, :] = mean
            rs_v[g, :] = y

        def pack_c(p):
            g = p // 4
            mus = []
            rss = []
            for r in range(4):
                gi = jnp.full((L,), g, jnp.int32)
                li = jnp.full((L,), (p % 4) * 4 + r, jnp.int32)
                mus.append(plsc.load_gather(mu_v, [gi, li]))
                rss.append(plsc.load_gather(rs_v, [gi, li]))

            def jbody(j):
                off = j * L
                gv = gam_v[pl.ds(off, L)]
                bv = bet_v[pl.ds(off, L)]
                for r in range(4):
                    row = p * 4 + r
                    h = hb[row, pl.ds(off, L)]
                    hb[row, pl.ds(off, L)] = (h - mus[r]) * rss[r] * gv + bv

            plsc.parallel_loop(0, D // L, unroll=UNR)(jbody)

        plsc.parallel_loop(0, PK)(pack_c)

    gh = {}
    ph = {}
    wh = {}
    gh[0], ph[0] = issue(0)
    for c in range(NCH):
        hs = c % 3
        ps = c % 2
        if c + 1 < NCH:
            if c - 2 >= 0:
                for h in wh[c - 2]:
                    h.wait()
            gh[c + 1], ph[c + 1] = issue(c + 1)
        gh[c].wait()
        ph[c].wait()
        compute_chunk(hs, ps)
        s0 = pl.multiple_of((base + c * C) // B, PC)
        wh[c] = [
            pltpu.async_copy(
                h_v.at[hs].at[pl.ds(r * B, B)],
                out_hbm.at[s0 + r], wsem[hs])
            for r in range(PC)
        ]
    for c in range(NCH - 3, NCH):
        for h in wh[c]:
            h.wait()


def kernel(x, tok_table, pos_table, gamma, beta):
    xf = x.reshape(NW, NCH, C).astype(jnp.int32)
    return _emb_ln(xf, tok_table, pos_table, gamma, beta)


# speedup vs baseline: 1.0133x; 1.0133x over previous
import functools

import jax
import jax.numpy as jnp
from jax import lax
from jax.experimental import pallas as pl
from jax.experimental.pallas import tpu as pltpu
from jax.experimental.pallas import tpu_sc as plsc

S = 2048
B = 4
D = 1024
N_TOK = S * B
L = 16
NC = 2
NS = 16
NW = NC * NS
TOK_PER_W = N_TOK // NW
C = 32
NCH = TOK_PER_W // C
PK = C // 4
NG = C // L
PC = C // B
UNR = 4
NJ = (D // L) // UNR

_MESH = plsc.VectorSubcoreMesh(core_axis_name="c", subcore_axis_name="s")


@functools.partial(
    pl.kernel,
    mesh=_MESH,
    compiler_params=pltpu.CompilerParams(needs_layout_passes=False),
    out_type=jax.ShapeDtypeStruct((S, B, D), jnp.float32),
    scratch_types=[
        pltpu.VMEM((NCH, C), jnp.int32),
        pltpu.VMEM((3, C, D), jnp.float32),
        pltpu.VMEM((2, PC, D), jnp.float32),
        pltpu.VMEM((D,), jnp.float32),
        pltpu.VMEM((D,), jnp.float32),
        pltpu.VMEM((C, L), jnp.float32),
        pltpu.VMEM((C, L), jnp.float32),
        pltpu.VMEM((NG, L), jnp.float32),
        pltpu.VMEM((NG, L), jnp.float32),
        pltpu.SemaphoreType.DMA,
        pltpu.SemaphoreType.DMA,
        pltpu.SemaphoreType.DMA,
        pltpu.SemaphoreType.DMA,
        pltpu.SemaphoreType.DMA,
        pltpu.SemaphoreType.DMA,
        pltpu.SemaphoreType.DMA,
        pltpu.SemaphoreType.DMA,
    ],
)
def _emb_ln(x_hbm, tok_hbm, pos_hbm, gamma_hbm, beta_hbm, out_hbm,
            idx_v, h_v, p_v, gam_v, bet_v, s_v, q_v, mu_v, rs_v,
            g0, g1, g2, p0, p1, w0, w1, w2):
    wid = lax.axis_index("s") * NC + lax.axis_index("c")
    base = wid * TOK_PER_W
    gsem = (g0, g1, g2)
    psem = (p0, p1)
    wsem = (w0, w1, w2)

    pltpu.sync_copy(gamma_hbm, gam_v)
    pltpu.sync_copy(beta_hbm, bet_v)
    pltpu.sync_copy(x_hbm.at[wid], idx_v)

    def issue(c):
        hs = c % 3
        ps = c % 2
        g = pltpu.async_copy(tok_hbm.at[idx_v.at[c]], h_v.at[hs], gsem[hs])
        pb = pl.multiple_of((base + c * C) // B, PC)
        p = pltpu.async_copy(pos_hbm.at[pl.ds(pb, PC)], p_v.at[ps], psem[ps])
        return g, p

    def compute_chunk(hs, ps):
        hb = h_v.at[hs]
        pb = p_v.at[ps]

        def pack_a(p):
            def jbody(j, accs):
                accs = list(accs)
                off = j * L
                pv = pb[p, pl.ds(off, L)]
                for r in range(4):
                    row = p * 4 + r
                    h = hb[row, pl.ds(off, L)] + pv
                    hb[row, pl.ds(off, L)] = h
                    accs[r] = accs[r] + h
                    accs[4 + r] = accs[4 + r] + h * h
                return tuple(accs)

            z = jnp.zeros((L,), jnp.float32)
            accs = plsc.parallel_loop(
                0, D // L, unroll=UNR, carry=(z,) * 8)(jbody)
            for r in range(4):
                s_v[p * 4 + r, :] = accs[r]
                q_v[p * 4 + r, :] = accs[4 + r]

        plsc.parallel_loop(0, PK)(pack_a)

        iota = lax.iota(jnp.int32, L)
        for g in range(NG):
            rows = g * L + iota
            tot = None
            tot2 = None
            for k in range(L):
                kk = jnp.full((L,), k, jnp.int32)
                cs = plsc.load_gather(s_v, [rows, kk])
                cq = plsc.load_gather(q_v, [rows, kk])
                tot = cs if tot is None else tot + cs
                tot2 = cq if tot2 is None else tot2 + cq
            mean = tot * (1.0 / D)
            var = tot2 * (1.0 / D) - mean * mean
            xe = var + 1e-5
            iv = lax.bitcast_convert_type(xe, jnp.int32)
            y = lax.bitcast_convert_type(
                jnp.full((L,), 0x5F3759DF, jnp.int32)
                - lax.shift_right_logical(iv, 1),
                jnp.float32)
            for _ in range(3):
                y = y * (1.5 - 0.5 * xe * y * y)
            mu_v[g, :] = mean
            rs_v[g, :] = y

        def pack_c(p):
            g = p // 4
            mus = []
            rss = []
            for r in range(4):
                gi = jnp.full((L,), g, jnp.int32)
                li = jnp.full((L,), (p % 4) * 4 + r, jnp.int32)
                mus.append(plsc.load_gather(mu_v, [gi, li]))
                rss.append(plsc.load_gather(rs_v, [gi, li]))

            def jbody(j):
                off = j * L
                gv = gam_v[pl.ds(off, L)]
                bv = bet_v[pl.ds(off, L)]
                for r in range(4):
                    row = p * 4 + r
                    h = hb[row, pl.ds(off, L)]
                    hb[row, pl.ds(off, L)] = (h - mus[r]) * rss[r] * gv + bv

            plsc.parallel_loop(0, D // L, unroll=UNR)(jbody)

        plsc.parallel_loop(0, PK)(pack_c)

    gh = {}
    ph = {}
    wh = {}
    gh[0], ph[0] = issue(0)
    for c in range(NCH):
        hs = c % 3
        ps = c % 2
        if c + 1 < NCH:
            if c - 2 >= 0:
                for h in wh[c - 2]:
                    h.wait()
            gh[c + 1], ph[c + 1] = issue(c + 1)
        gh[c].wait()
        ph[c].wait()
        compute_chunk(hs, ps)
        s0 = pl.multiple_of((base + c * C) // B, PC)
        wh[c] = [
            pltpu.async_copy(
                h_v.at[hs].at[pl.ds(r * B, B)],
                out_hbm.at[s0 + r], wsem[hs])
            for r in range(PC)
        ]
    for c in range(NCH - 3, NCH):
        for h in wh[c]:
            h.wait()


def kernel(x, tok_table, pos_table, gamma, beta):
    xf = x.reshape(NW, NCH, C).astype(jnp.int32)
    return _emb_ln(xf, tok_table, pos_table, gamma, beta)
